# baseline (speedup 1.0000x reference)
import functools

import jax
import jax.numpy as jnp
from jax import lax
from jax.experimental import pallas as pl
from jax.experimental.pallas import tpu as pltpu
from jax.experimental.pallas import tpu_sc as plsc

D_MODEL = 64
CHUNK = 128
NUM_WORKERS = 32


def _make_gather_kernel(n_rows: int, d: int):
    b_per_w = n_rows // NUM_WORKERS
    n_chunks = b_per_w // CHUNK
    assert n_chunks % 2 == 0
    mesh = plsc.VectorSubcoreMesh(core_axis_name="c", subcore_axis_name="s")

    @functools.partial(
        pl.kernel,
        mesh=mesh,
        compiler_params=pltpu.CompilerParams(use_tc_tiling_on_sc=False),
        out_type=jax.ShapeDtypeStruct((n_rows, d), jnp.float32),
        scratch_types=[
            pltpu.VMEM((n_chunks, CHUNK), jnp.int32),
            pltpu.VMEM((CHUNK, d), jnp.float32),
            pltpu.VMEM((CHUNK, d), jnp.float32),
            pltpu.SemaphoreType.DMA,
            pltpu.SemaphoreType.DMA,
        ],
    )
    def gather_kernel(table_hbm, idx_hbm, out_hbm, idx_v, buf0, buf1, sem0, sem1):
        wid = lax.axis_index("s") * 2 + lax.axis_index("c")
        chunk_base = wid * n_chunks
        row_base = wid * b_per_w
        pltpu.sync_copy(idx_hbm.at[pl.ds(chunk_base, n_chunks)], idx_v)
        pltpu.async_copy(table_hbm.at[idx_v.at[0]], buf0, sem0)

        def pair(h, carry):
            j0 = 2 * h
            pltpu.async_copy(table_hbm.at[idx_v.at[j0 + 1]], buf1, sem1)
            pltpu.make_async_copy(table_hbm.at[idx_v.at[j0]], buf0, sem0).wait()
            pltpu.sync_copy(buf0, out_hbm.at[pl.ds(row_base + j0 * CHUNK, CHUNK)])

            @pl.when(j0 + 2 < n_chunks)
            def _():
                pltpu.async_copy(table_hbm.at[idx_v.at[j0 + 2]], buf0, sem0)

            pltpu.make_async_copy(table_hbm.at[idx_v.at[j0 + 1]], buf1, sem1).wait()
            pltpu.sync_copy(
                buf1, out_hbm.at[pl.ds(row_base + (j0 + 1) * CHUNK, CHUNK)]
            )
            return carry

        lax.fori_loop(0, n_chunks // 2, pair, 0)

    return gather_kernel


_gather = _make_gather_kernel(4096 * 200, D_MODEL)


@jax.jit
def kernel(x, table):
    idx = x.reshape(-1, CHUNK).astype(jnp.int32)
    out = _gather(table, idx)
    return out.reshape(x.shape + (D_MODEL,))


# speedup vs baseline: 1.0194x; 1.0194x over previous
import functools

import jax
import jax.numpy as jnp
from jax import lax
from jax.experimental import pallas as pl
from jax.experimental.pallas import tpu as pltpu
from jax.experimental.pallas import tpu_sc as plsc

D_MODEL = 64
CHUNK = 128
NUM_WORKERS = 32
NBUF = 8
LOOKAHEAD = 4


def _make_gather_kernel(n_rows: int, d: int):
    b_per_w = n_rows // NUM_WORKERS
    n_chunks = b_per_w // CHUNK
    n_groups = n_chunks // NBUF
    assert n_chunks % NBUF == 0 and n_groups >= 3
    mesh = plsc.VectorSubcoreMesh(core_axis_name="c", subcore_axis_name="s")

    scratch = [pltpu.VMEM((n_chunks, CHUNK), jnp.int32)]
    scratch += [pltpu.VMEM((CHUNK, d), jnp.float32) for _ in range(NBUF)]
    scratch += [pltpu.SemaphoreType.DMA for _ in range(2 * NBUF)]

    @functools.partial(
        pl.kernel,
        mesh=mesh,
        compiler_params=pltpu.CompilerParams(use_tc_tiling_on_sc=False),
        out_type=jax.ShapeDtypeStruct((n_rows, d), jnp.float32),
        scratch_types=scratch,
    )
    def gather_kernel(table_hbm, idx_hbm, out_hbm, idx_v, *rest):
        bufs = rest[:NBUF]
        gsem = rest[NBUF : 2 * NBUF]
        osem = rest[2 * NBUF :]
        wid = lax.axis_index("s") * 2 + lax.axis_index("c")
        chunk_base = wid * n_chunks
        row_base = wid * b_per_w
        pltpu.sync_copy(idx_hbm.at[pl.ds(chunk_base, n_chunks)], idx_v)

        def issue_gather(c, b):
            pltpu.async_copy(table_hbm.at[idx_v.at[c]], bufs[b], gsem[b])

        def wait_gather(c, b):
            pltpu.make_async_copy(table_hbm.at[idx_v.at[c]], bufs[b], gsem[b]).wait()

        def out_ref(c):
            return out_hbm.at[pl.ds(row_base + c * CHUNK, CHUNK)]

        def issue_out(c, b):
            pltpu.async_copy(bufs[b], out_ref(c), osem[b])

        def wait_out(c, b):
            pltpu.make_async_copy(bufs[b], out_ref(c), osem[b]).wait()

        for b in range(LOOKAHEAD):
            issue_gather(b, b)

        def stage(c, b, *, skip_out_wait=False, skip_issue=False):
            wait_gather(c, b)
            issue_out(c, b)
            if not skip_issue:
                bj = (b + LOOKAHEAD) % NBUF
                if not skip_out_wait:
                    wait_out(c + LOOKAHEAD - NBUF, bj)
                issue_gather(c + LOOKAHEAD, bj)

        for b in range(NBUF):
            stage(b, b, skip_out_wait=b + LOOKAHEAD < NBUF)

        def group(i, carry):
            c0 = i * NBUF
            for b in range(NBUF):
                stage(c0 + b, b)
            return carry

        lax.fori_loop(1, n_groups - 1, group, 0)

        c0 = n_chunks - NBUF
        for b in range(NBUF):
            stage(c0 + b, b, skip_issue=b >= NBUF - LOOKAHEAD)

        for b in range(NBUF):
            wait_out(n_chunks - NBUF + b, b)

    return gather_kernel


_gather = _make_gather_kernel(4096 * 200, D_MODEL)


@jax.jit
def kernel(x, table):
    idx = x.reshape(-1, CHUNK).astype(jnp.int32)
    out = _gather(table, idx)
    return out.reshape(x.shape + (D_MODEL,))
